# Initial kernel scaffold; baseline (speedup 1.0000x reference)
import jax
import jax.numpy as jnp
from jax.experimental import pallas as pl


def kernel(x, adj, W1, b1, W2, b2):
    raise NotImplementedError("write your pallas kernel here")


# speedup vs baseline: 1.1785x; 1.1785x over previous
import functools

import jax
import jax.numpy as jnp
from jax.experimental import pallas as pl
from jax.experimental.pallas import tpu as pltpu


def _proj_kernel(x_ref, w_ref, b_ref, o_ref):
    o_ref[...] = (
        jnp.dot(x_ref[...], w_ref[...], preferred_element_type=jnp.float32)
        + b_ref[...]
    )


def _proj(x, w, b, bm):
    n, f = x.shape
    fo = w.shape[1]
    return pl.pallas_call(
        _proj_kernel,
        grid=(n // bm,),
        in_specs=[
            pl.BlockSpec((bm, f), lambda i: (i, 0)),
            pl.BlockSpec((f, fo), lambda i: (0, 0)),
            pl.BlockSpec((1, fo), lambda i: (0, 0)),
        ],
        out_specs=pl.BlockSpec((bm, fo), lambda i: (i, 0)),
        out_shape=jax.ShapeDtypeStruct((n, fo), jnp.float32),
    )(x, w, b.reshape(1, fo))


def _agg_kernel(adj_ref, y_ref, base_ref, o_ref, *, sigmoid):
    a = adj_ref[...]
    deg = jnp.sum(a, axis=1, keepdims=True) + 1e-8
    r = base_ref[...] + jnp.dot(a, y_ref[...], preferred_element_type=jnp.float32) / deg
    if sigmoid:
        r = jax.nn.sigmoid(r)
    o_ref[...] = r


def _agg(adj, y, base, bm, sigmoid):
    n = adj.shape[0]
    fo = y.shape[1]
    return pl.pallas_call(
        functools.partial(_agg_kernel, sigmoid=sigmoid),
        grid=(n // bm,),
        in_specs=[
            pl.BlockSpec((bm, n), lambda i: (i, 0)),
            pl.BlockSpec((n, fo), lambda i: (0, 0)),
            pl.BlockSpec((bm, fo), lambda i: (i, 0)),
        ],
        out_specs=pl.BlockSpec((bm, fo), lambda i: (i, 0)),
        out_shape=jax.ShapeDtypeStruct((n, fo), jnp.float32),
        compiler_params=pltpu.CompilerParams(
            dimension_semantics=("parallel",),
        ),
    )(adj, y, base)


def kernel(x, adj, W1, b1, W2, b2):
    n, f = x.shape
    h1 = W1.shape[1]
    c = W2.shape[1]
    bm = 400 if n % 400 == 0 else n

    wc1 = jnp.concatenate([W1[:f], W1[f:]], axis=1)
    bc1 = jnp.concatenate([b1, jnp.zeros_like(b1)])
    p1 = _proj(x, wc1, bc1, bm)
    h = _agg(adj, p1[:, h1:], p1[:, :h1], bm, sigmoid=False)

    wc2 = jnp.concatenate([W2[:h1], W2[h1:]], axis=1)
    bc2 = jnp.concatenate([b2, jnp.zeros_like(b2)])
    p2 = _proj(h, wc2, bc2, bm)
    return _agg(adj, p2[:, c:], p2[:, :c], bm, sigmoid=True)


# speedup vs baseline: 1.3024x; 1.1051x over previous
import functools

import jax
import jax.numpy as jnp
from jax.experimental import pallas as pl
from jax.experimental.pallas import tpu as pltpu


def _proj1_kernel(x_ref, w_ref, b_ref, o1_ref, o2_ref):
    h1 = o1_ref.shape[1]
    r = (
        jnp.dot(x_ref[...], w_ref[...], preferred_element_type=jnp.float32)
        + b_ref[...]
    )
    o1_ref[...] = r[:, :h1]
    o2_ref[...] = r[:, h1:].astype(jnp.bfloat16)


def _proj1(x, w, b, bm):
    n, f = x.shape
    fo = w.shape[1]
    h1 = fo // 2
    return pl.pallas_call(
        _proj1_kernel,
        grid=(n // bm,),
        in_specs=[
            pl.BlockSpec((bm, f), lambda i: (i, 0)),
            pl.BlockSpec((f, fo), lambda i: (0, 0)),
            pl.BlockSpec((1, fo), lambda i: (0, 0)),
        ],
        out_specs=[
            pl.BlockSpec((bm, h1), lambda i: (i, 0)),
            pl.BlockSpec((bm, h1), lambda i: (i, 0)),
        ],
        out_shape=[
            jax.ShapeDtypeStruct((n, h1), jnp.float32),
            jax.ShapeDtypeStruct((n, h1), jnp.bfloat16),
        ],
    )(x, w, b.reshape(1, fo))


def _agg1_kernel(adj_ref, y_ref, base_ref, w_ref, b_ref, o1_ref, o2_ref):
    c = o1_ref.shape[1]
    a = adj_ref[...]
    deg = jnp.sum(a, axis=1, keepdims=True) + 1e-8
    dot = jnp.dot(
        a.astype(jnp.bfloat16), y_ref[...], preferred_element_type=jnp.float32
    )
    h = base_ref[...] + dot / deg
    p2 = jnp.dot(h, w_ref[...], preferred_element_type=jnp.float32) + b_ref[...]
    o1_ref[...] = p2[:, :c]
    o2_ref[...] = p2[:, c:].astype(jnp.bfloat16)


def _agg1(adj, y, base, w, b, bm):
    n = adj.shape[0]
    h1 = y.shape[1]
    fo = w.shape[1]
    c = fo // 2
    return pl.pallas_call(
        _agg1_kernel,
        grid=(n // bm,),
        in_specs=[
            pl.BlockSpec((bm, n), lambda i: (i, 0)),
            pl.BlockSpec((n, h1), lambda i: (0, 0)),
            pl.BlockSpec((bm, h1), lambda i: (i, 0)),
            pl.BlockSpec((h1, fo), lambda i: (0, 0)),
            pl.BlockSpec((1, fo), lambda i: (0, 0)),
        ],
        out_specs=[
            pl.BlockSpec((bm, c), lambda i: (i, 0)),
            pl.BlockSpec((bm, c), lambda i: (i, 0)),
        ],
        out_shape=[
            jax.ShapeDtypeStruct((n, c), jnp.float32),
            jax.ShapeDtypeStruct((n, c), jnp.bfloat16),
        ],
        compiler_params=pltpu.CompilerParams(
            dimension_semantics=("parallel",),
        ),
    )(adj, y, base, w, b.reshape(1, fo))


def _agg2_kernel(adj_ref, y_ref, base_ref, o_ref):
    a = adj_ref[...]
    deg = jnp.sum(a, axis=1, keepdims=True) + 1e-8
    dot = jnp.dot(
        a.astype(jnp.bfloat16), y_ref[...], preferred_element_type=jnp.float32
    )
    o_ref[...] = jax.nn.sigmoid(base_ref[...] + dot / deg)


def _agg2(adj, y, base, bm):
    n = adj.shape[0]
    c = y.shape[1]
    return pl.pallas_call(
        _agg2_kernel,
        grid=(n // bm,),
        in_specs=[
            pl.BlockSpec((bm, n), lambda i: (i, 0)),
            pl.BlockSpec((n, c), lambda i: (0, 0)),
            pl.BlockSpec((bm, c), lambda i: (i, 0)),
        ],
        out_specs=pl.BlockSpec((bm, c), lambda i: (i, 0)),
        out_shape=jax.ShapeDtypeStruct((n, c), jnp.float32),
        compiler_params=pltpu.CompilerParams(
            dimension_semantics=("parallel",),
        ),
    )(adj, y, base)


def kernel(x, adj, W1, b1, W2, b2):
    n, f = x.shape
    h1 = W1.shape[1]
    bm = 400 if n % 400 == 0 else n

    wc1 = jnp.concatenate([W1[:f], W1[f:]], axis=1)
    bc1 = jnp.concatenate([b1, jnp.zeros_like(b1)])
    xw1, y1 = _proj1(x, wc1, bc1, bm)

    wc2 = jnp.concatenate([W2[:h1], W2[h1:]], axis=1)
    bc2 = jnp.concatenate([b2, jnp.zeros_like(b2)])
    hw2, y2 = _agg1(adj, y1, xw1, wc2, bc2, bm)

    return _agg2(adj, y2, hw2, bm)


# speedup vs baseline: 1.4371x; 1.1034x over previous
import functools

import jax
import jax.numpy as jnp
from jax.experimental import pallas as pl
from jax.experimental.pallas import tpu as pltpu


def _sage_kernel(
    adj_ref,
    x_ref,
    wt1_ref,
    wb1_ref,
    b1_ref,
    wc2_ref,
    bc2_ref,
    o_ref,
    y1_ref,
    hw2_ref,
    y2_ref,
):
    t = pl.program_id(0)
    i = pl.program_id(1)
    bm = adj_ref.shape[0]
    c = o_ref.shape[1]

    @pl.when((t == 0) & (i == 0))
    def _():
        y1_ref[...] = jnp.dot(
            x_ref[...], wb1_ref[...], preferred_element_type=jnp.float32
        ).astype(jnp.bfloat16)

    a = adj_ref[...]
    deg = jnp.sum(a, axis=1, keepdims=True) + 1e-8
    ab = a.astype(jnp.bfloat16)
    rows = pl.ds(i * bm, bm)

    @pl.when(t == 0)
    def _():
        agg = jnp.dot(ab, y1_ref[...], preferred_element_type=jnp.float32) / deg
        h = (
            jnp.dot(x_ref[rows, :], wt1_ref[...], preferred_element_type=jnp.float32)
            + b1_ref[...]
            + agg
        )
        p2 = (
            jnp.dot(h, wc2_ref[...], preferred_element_type=jnp.float32)
            + bc2_ref[...]
        )
        hw2_ref[rows, :] = p2[:, :c]
        y2_ref[rows, :] = p2[:, c:].astype(jnp.bfloat16)

    @pl.when(t == 1)
    def _():
        agg = jnp.dot(ab, y2_ref[...], preferred_element_type=jnp.float32) / deg
        o_ref[...] = jax.nn.sigmoid(hw2_ref[rows, :] + agg)


def kernel(x, adj, W1, b1, W2, b2):
    n, f = x.shape
    h1 = W1.shape[1]
    c = W2.shape[1]
    bm = 400 if n % 400 == 0 else n

    wt1 = W1[:f]
    wb1 = W1[f:]
    wc2 = jnp.concatenate([W2[:h1], W2[h1:]], axis=1)
    bc2 = jnp.concatenate([b2, jnp.zeros_like(b2)]).reshape(1, 2 * c)

    return pl.pallas_call(
        _sage_kernel,
        grid=(2, n // bm),
        in_specs=[
            pl.BlockSpec((bm, n), lambda t, i: (i, 0)),
            pl.BlockSpec((n, f), lambda t, i: (0, 0)),
            pl.BlockSpec((f, h1), lambda t, i: (0, 0)),
            pl.BlockSpec((f, h1), lambda t, i: (0, 0)),
            pl.BlockSpec((1, h1), lambda t, i: (0, 0)),
            pl.BlockSpec((h1, 2 * c), lambda t, i: (0, 0)),
            pl.BlockSpec((1, 2 * c), lambda t, i: (0, 0)),
        ],
        out_specs=pl.BlockSpec((bm, c), lambda t, i: (i, 0)),
        out_shape=jax.ShapeDtypeStruct((n, c), jnp.float32),
        scratch_shapes=[
            pltpu.VMEM((n, h1), jnp.bfloat16),
            pltpu.VMEM((n, c), jnp.float32),
            pltpu.VMEM((n, c), jnp.bfloat16),
        ],
        compiler_params=pltpu.CompilerParams(
            dimension_semantics=("arbitrary", "arbitrary"),
        ),
    )(adj, x, wt1, wb1, b1.reshape(1, h1), wc2, bc2)
